# baseline (speedup 1.0000x reference)
import functools

import jax
import jax.numpy as jnp
from jax.experimental import pallas as pl
from jax.experimental.pallas import tpu as pltpu

_BLOCK_ROWS = 512
_RANK_PAD = 128


def _fused_lowrank_kernel(x_ref, win_ref, wout_ref, out_ref):
    x = x_ref[...].astype(jnp.bfloat16)
    h = jnp.dot(x, win_ref[...].astype(jnp.bfloat16),
                preferred_element_type=jnp.float32)
    out_ref[...] = jnp.dot(h.astype(jnp.bfloat16),
                           wout_ref[...].astype(jnp.bfloat16),
                           preferred_element_type=jnp.float32)


@functools.partial(jax.jit, static_argnames=())
def kernel(x, shared_in, shared_out):
    n_tokens, in_features = x.shape
    rank, out_features = shared_out.shape

    win = jnp.zeros((in_features, _RANK_PAD), dtype=shared_in.dtype)
    win = win.at[:, :rank].set(shared_in)
    wout = jnp.zeros((_RANK_PAD, out_features), dtype=shared_out.dtype)
    wout = wout.at[:rank, :].set(shared_out)

    grid = (n_tokens // _BLOCK_ROWS,)
    return pl.pallas_call(
        _fused_lowrank_kernel,
        grid=grid,
        in_specs=[
            pl.BlockSpec((_BLOCK_ROWS, in_features), lambda i: (i, 0)),
            pl.BlockSpec((in_features, _RANK_PAD), lambda i: (0, 0)),
            pl.BlockSpec((_RANK_PAD, out_features), lambda i: (0, 0)),
        ],
        out_specs=pl.BlockSpec((_BLOCK_ROWS, out_features), lambda i: (i, 0)),
        out_shape=jax.ShapeDtypeStruct((n_tokens, out_features), jnp.float32),
        compiler_params=pltpu.CompilerParams(
            dimension_semantics=("parallel",),
        ),
    )(x, win, wout)


# speedup vs baseline: 1.1014x; 1.1014x over previous
import functools

import jax
import jax.numpy as jnp
from jax.experimental import pallas as pl
from jax.experimental.pallas import tpu as pltpu

_BLOCK_ROWS = 1024
_RANK_PAD = 128


def _fused_lowrank_kernel(x_ref, win_ref, wout_ref, out_ref):
    h = jnp.dot(x_ref[...].astype(jnp.bfloat16), win_ref[...],
                preferred_element_type=jnp.float32)
    out_ref[...] = jnp.dot(h.astype(jnp.bfloat16), wout_ref[...],
                           preferred_element_type=jnp.float32)


@functools.partial(jax.jit, static_argnames=())
def kernel(x, shared_in, shared_out):
    n_tokens, in_features = x.shape
    rank, out_features = shared_out.shape

    win = jnp.zeros((in_features, _RANK_PAD), dtype=jnp.bfloat16)
    win = win.at[:, :rank].set(shared_in.astype(jnp.bfloat16))
    wout = jnp.zeros((_RANK_PAD, out_features), dtype=jnp.bfloat16)
    wout = wout.at[:rank, :].set(shared_out.astype(jnp.bfloat16))

    grid = (n_tokens // _BLOCK_ROWS,)
    return pl.pallas_call(
        _fused_lowrank_kernel,
        grid=grid,
        in_specs=[
            pl.BlockSpec((_BLOCK_ROWS, in_features), lambda i: (i, 0)),
            pl.BlockSpec((in_features, _RANK_PAD), lambda i: (0, 0)),
            pl.BlockSpec((_RANK_PAD, out_features), lambda i: (0, 0)),
        ],
        out_specs=pl.BlockSpec((_BLOCK_ROWS, out_features), lambda i: (i, 0)),
        out_shape=jax.ShapeDtypeStruct((n_tokens, out_features), jnp.float32),
        compiler_params=pltpu.CompilerParams(
            dimension_semantics=("parallel",),
        ),
    )(x, win, wout)


# speedup vs baseline: 1.1367x; 1.0320x over previous
import jax
import jax.numpy as jnp
from jax.experimental import pallas as pl
from jax.experimental.pallas import tpu as pltpu

_BLOCK_ROWS = 1024


def _fused_lowrank_kernel(x_ref, win_ref, wout_ref, out_ref):
    h = jnp.dot(x_ref[...], win_ref[...], preferred_element_type=jnp.float32)
    out_ref[...] = jnp.dot(h, wout_ref[...], preferred_element_type=jnp.float32)


@jax.jit
def kernel(x, shared_in, shared_out):
    n_tokens, in_features = x.shape
    rank, out_features = shared_out.shape

    grid = (n_tokens // _BLOCK_ROWS,)
    return pl.pallas_call(
        _fused_lowrank_kernel,
        grid=grid,
        in_specs=[
            pl.BlockSpec((_BLOCK_ROWS, in_features), lambda i: (i, 0)),
            pl.BlockSpec((in_features, rank), lambda i: (0, 0)),
            pl.BlockSpec((rank, out_features), lambda i: (0, 0)),
        ],
        out_specs=pl.BlockSpec((_BLOCK_ROWS, out_features), lambda i: (i, 0)),
        out_shape=jax.ShapeDtypeStruct((n_tokens, out_features), jnp.float32),
        compiler_params=pltpu.CompilerParams(
            dimension_semantics=("parallel",),
        ),
    )(x, shared_in, shared_out)
